# baseline (speedup 1.0000x reference)
import functools

import jax
import jax.numpy as jnp
from jax import lax
from jax.experimental import pallas as pl
from jax.experimental.pallas import tpu as pltpu
from jax.experimental.pallas import tpu_sc as plsc

VOCAB = 1000000
EMBED_DIM = 16
BATCH = 16384
FIELDS = 26
N = BATCH * FIELDS
NUM_CORES = 2
NUM_SUBCORES = 16
NW = NUM_CORES * NUM_SUBCORES
GW = 128
G = N // GW
G_PER_W = G // NW
K = 13
NCHUNK = G_PER_W // K


def _make_kernel():
    mesh = plsc.VectorSubcoreMesh(core_axis_name="c", subcore_axis_name="s")

    @functools.partial(
        pl.kernel,
        mesh=mesh,
        out_type=jax.ShapeDtypeStruct((G, GW, EMBED_DIM), jnp.float32),
        scratch_types=[
            pltpu.VMEM((G_PER_W, GW), jnp.int32),
            pltpu.VMEM((K, GW, EMBED_DIM), jnp.float32),
            pltpu.SemaphoreType.DMA,
        ],
        compiler_params=pltpu.CompilerParams(use_tc_tiling_on_sc=False),
    )
    def k(idx_hbm, table_hbm, out_hbm, idx_v, rows_v, sem):
        wid = lax.axis_index("s") * NUM_CORES + lax.axis_index("c")
        gbase = wid * G_PER_W
        pltpu.sync_copy(idx_hbm.at[pl.ds(gbase, G_PER_W)], idx_v)

        def chunk(c, carry):
            copies = [
                pltpu.async_copy(
                    table_hbm.at[idx_v.at[c * K + j]], rows_v.at[j], sem)
                for j in range(K)
            ]
            for cp in copies:
                cp.wait()
            pltpu.sync_copy(rows_v, out_hbm.at[pl.ds(gbase + c * K, K)])
            return carry

        lax.fori_loop(0, NCHUNK, chunk, 0)

    return k


_embed_gather = _make_kernel()


def kernel(x, table):
    idx = x.reshape(G, GW).astype(jnp.int32)
    out = _embed_gather(idx, table)
    return out.reshape(BATCH, FIELDS, EMBED_DIM)
